# baseline (speedup 1.0000x reference)
import functools

import jax
import jax.numpy as jnp
from jax import lax
from jax.experimental import pallas as pl
from jax.experimental.pallas import tpu as pltpu
from jax.experimental.pallas import tpu_sc as plsc

_V = 1000000
_D = 32
_L = 50
_B = 16384
_LANES = 16

_NC = 2
_NS = 16
_CPC = _D // _NC
_BPS = _B // _NS
_NG = _BPS // _LANES

_SEG = 62464
_CHK = 1024
_NCHK = _SEG // _CHK
_SEG_EXTRA_OFF = _SEG * _NS
_SEG_EXTRA = 512
_TAIL_OFF = _SEG_EXTRA_OFF + _SEG_EXTRA
_TAIL = _V - _TAIL_OFF


def _sc_body(tabt_hbm, enct_hbm, tail_hbm, w_hbm, b_hbm, zero_hbm, outt_hbm,
             col0, enc_v, dst0, dst1, w_v, b_v, tail_v, tmp64,
             cidx_v, lidx_v,
             out0, out1, sem_i, sem_g0, sem_g1, sem_o0, sem_o1,
             sem_bg0, sem_bg1, sem_bp0, sem_bp1):
    core = lax.axis_index("c")
    sid = lax.axis_index("s")
    b0 = sid * _BPS
    c_base = core * _CPC
    seg_off = sid * _SEG

    pltpu.sync_copy(w_hbm, w_v)
    pltpu.sync_copy(b_hbm, b_v)
    pltpu.sync_copy(tail_hbm, tail_v)

    for k in range(_D):
        cidx_v[pl.ds(8 * k, _LANES)] = jnp.broadcast_to(
            jnp.int32(k), (_LANES,))
    for k in range(_L):
        lidx_v[pl.ds(8 * k, _LANES)] = jnp.broadcast_to(
            jnp.int32(k), (_LANES,))

    def cidx_at(c):
        return cidx_v.at[pl.ds(8 * c, 1)]

    def lidx_at(lv):
        return lidx_v.at[pl.ds(8 * lv, 1)]

    dsts = (dst0, dst1)
    sem_g = (sem_g0, sem_g1)
    bbs = (dst0, dst1)
    sem_bg = (sem_bg0, sem_bg1)
    sem_bp = (sem_bp0, sem_bp1)
    outs = ((out0, sem_o0), (out1, sem_o1))

    def stage_enc(lv, carry):
        pltpu.async_copy(
            enct_hbm.at[lidx_at(lv), pl.ds(b0, _BPS)],
            enc_v.at[pl.ds(0, 1), pl.ds(lv * _BPS, _BPS)],
            sem_i)
        return carry

    def drain_enc(lv, carry):
        pltpu.make_async_copy(
            zero_hbm, enc_v.at[0, pl.ds(lv * _BPS, _BPS)], sem_i).wait()
        return carry

    def bounce_gather(c, k, i):
        pltpu.async_copy(
            tabt_hbm.at[cidx_at(c), pl.ds(seg_off + k * _CHK, _CHK)],
            bbs[i],
            sem_bg[i])

    def wait_bounce_gather(i):
        pltpu.make_async_copy(
            col0.at[pl.ds(0, _CHK)], bbs[i].at[0], sem_bg[i]).wait()

    def bounce_push(k, i):
        pltpu.async_copy(
            bbs[i].at[0], col0.at[pl.ds(seg_off + k * _CHK, _CHK)], sem_bp[i])

    def wait_bounce_push(i):
        pltpu.make_async_copy(
            bbs[i].at[0], col0.at[pl.ds(seg_off, _CHK)], sem_bp[i]).wait()

    def stage_col(c):
        cg = c_base + c
        bounce_gather(cg, 0, 0)
        bounce_gather(cg, 1, 1)

        def chunk_pair(q, carry):
            k = 2 * q
            for i in range(2):
                wait_bounce_gather(i)
                bounce_push(k + i, i)

                @pl.when(k + i + 2 < _NCHK)
                def _():
                    wait_bounce_push(i)
                    bounce_gather(cg, k + i + 2, i)

            return carry

        lax.fori_loop(0, _NCHK // 2, chunk_pair, 0)
        wait_bounce_gather(0)
        bounce_push(_NCHK - 1, 0)

        @pl.when(sid == 1)
        def _():
            wait_bounce_push(1)
            pltpu.async_copy(
                tabt_hbm.at[cidx_at(cg), pl.ds(_SEG_EXTRA_OFF, _SEG_EXTRA)],
                bbs[1].at[pl.ds(0, 1), pl.ds(0, _SEG_EXTRA)],
                sem_bg[1])
            pltpu.make_async_copy(
                col0.at[pl.ds(0, _SEG_EXTRA)],
                bbs[1].at[0, pl.ds(0, _SEG_EXTRA)], sem_bg[1]).wait()
            pltpu.async_copy(
                bbs[1].at[0, pl.ds(0, _SEG_EXTRA)],
                col0.at[pl.ds(_SEG_EXTRA_OFF, _SEG_EXTRA)],
                sem_bp[1])

        @pl.when(sid == 0)
        def _():
            for k in range(_TAIL // _LANES):
                tmp64[pl.ds(k * _LANES, _LANES)] = \
                    tail_v[pl.ds((c_base + c) * _TAIL + k * _LANES, _LANES)]
            pltpu.sync_copy(tmp64, col0.at[pl.ds(_TAIL_OFF, _TAIL)])

        wait_bounce_push(0)

        @pl.when(sid != 1)
        def _():
            wait_bounce_push(1)

        @pl.when(sid == 1)
        def _():
            pltpu.make_async_copy(
                bbs[1].at[0, pl.ds(0, _SEG_EXTRA)],
                col0.at[pl.ds(_SEG_EXTRA_OFF, _SEG_EXTRA)],
                sem_bp[1]).wait()

    def issue_lookup(lv, i):
        pltpu.async_copy(
            col0.at[enc_v.at[0, pl.ds(lv * _BPS, _BPS)],],
            dsts[i].at[0],
            sem_g[i])

    def wait_lookup(i):
        pltpu.make_async_copy(
            col0.at[pl.ds(0, _BPS)], dsts[i].at[0], sem_g[i]).wait()

    def lookup_and_compute(out_v):
        def init(g, carry):
            out_v[0, pl.ds(g * _LANES, _LANES)] = b_v[:]
            return carry

        lax.fori_loop(0, _NG, init, 0)

        issue_lookup(0, 0)
        issue_lookup(1, 1)

        def lpair(q, carry):
            for i in range(2):
                lv = 2 * q + i
                wait_lookup(i)

                def accum(g, carry2):
                    off = g * _LANES
                    out_v[0, pl.ds(off, _LANES)] = (
                        out_v[0, pl.ds(off, _LANES)]
                        + w_v[pl.ds(lv * _LANES, _LANES)]
                        * dsts[i][0, pl.ds(off, _LANES)])
                    return carry2

                lax.fori_loop(0, _NG, accum, 0)

                @pl.when(lv + 2 < _L)
                def _():
                    issue_lookup(lv + 2, i)

            return carry

        lax.fori_loop(0, _L // 2, lpair, 0)

        def relu(g, carry):
            off = g * _LANES
            out_v[0, pl.ds(off, _LANES)] = jnp.maximum(
                out_v[0, pl.ds(off, _LANES)], 0.0)
            return carry

        lax.fori_loop(0, _NG, relu, 0)

    lax.fori_loop(0, _L, stage_enc, 0)
    lax.fori_loop(0, _L, drain_enc, 0)
    stage_col(0)
    plsc.subcore_barrier()

    def body(half, carry):
        for par in range(2):
            c = 2 * half + par
            out_v, sem_o = outs[par]

            @pl.when(c >= 2)
            def _():
                pltpu.make_async_copy(
                    out_v,
                    outt_hbm.at[cidx_at(0), pl.ds(b0, _BPS)],
                    sem_o).wait()

            lookup_and_compute(out_v)
            pltpu.async_copy(
                out_v, outt_hbm.at[cidx_at(c_base + c), pl.ds(b0, _BPS)],
                sem_o)

            plsc.subcore_barrier()

            @pl.when(c + 1 < _CPC)
            def _():
                stage_col(c + 1)

            plsc.subcore_barrier()
        return carry

    lax.fori_loop(0, _CPC // 2, body, 0)

    for par in range(2):
        out_v, sem_o = outs[par]
        pltpu.make_async_copy(
            out_v,
            outt_hbm.at[cidx_at(0), pl.ds(b0, _BPS)],
            sem_o).wait()


@jax.jit
def _run(enct, tabt, tail_t, w16, b16):
    mesh = plsc.VectorSubcoreMesh(core_axis_name="c", subcore_axis_name="s")
    sc = functools.partial(
        pl.kernel,
        out_type=jax.ShapeDtypeStruct((_D, _B), jnp.float32),
        mesh=mesh,
        scratch_types=[
            pltpu.VMEM_SHARED((_V,), jnp.float32),
            pltpu.VMEM((1, _L * _BPS), jnp.int32),
            pltpu.VMEM((1, _BPS), jnp.float32),
            pltpu.VMEM((1, _BPS), jnp.float32),
            pltpu.VMEM((_L * _LANES,), jnp.float32),
            pltpu.VMEM((_LANES,), jnp.float32),
            pltpu.VMEM((_D * _TAIL,), jnp.float32),
            pltpu.VMEM((_TAIL,), jnp.float32),
            pltpu.VMEM((8 * _D + 8, ), jnp.int32),
            pltpu.VMEM((8 * _L + 8, ), jnp.int32),
            pltpu.VMEM((1, _BPS), jnp.float32),
            pltpu.VMEM((1, _BPS), jnp.float32),
            pltpu.SemaphoreType.DMA,
            pltpu.SemaphoreType.DMA,
            pltpu.SemaphoreType.DMA,
            pltpu.SemaphoreType.DMA,
            pltpu.SemaphoreType.DMA,
            pltpu.SemaphoreType.DMA,
            pltpu.SemaphoreType.DMA,
            pltpu.SemaphoreType.DMA,
            pltpu.SemaphoreType.DMA,
        ],
    )(_sc_body)
    zero = jnp.zeros((_BPS,), jnp.int32)
    return sc(tabt, enct, tail_t, w16, b16, zero)


def kernel(encoded, table, W, b):
    enct = jnp.transpose(encoded.astype(jnp.int32))
    tabt = jnp.transpose(table)
    tail_t = jnp.transpose(table[_TAIL_OFF:, :]).reshape(-1)
    w16 = jnp.broadcast_to(
        W.astype(jnp.float32).reshape(_L, 1), (_L, _LANES)).reshape(-1)
    b16 = jnp.broadcast_to(b.astype(jnp.float32).reshape(1), (_LANES,))
    outt = _run(enct, tabt, tail_t, w16, b16)
    return jnp.transpose(outt)


# speedup vs baseline: 1.2036x; 1.2036x over previous
import functools

import jax
import jax.numpy as jnp
from jax import lax
from jax.experimental import pallas as pl
from jax.experimental.pallas import tpu as pltpu
from jax.experimental.pallas import tpu_sc as plsc

_D = 32
_L = 50
_B = 16384
_LANES = 16

_NC = 2
_NS = 16
_NW = _NC * _NS

_ROWS_PER_W = _B // _NW
_CB = 32
_NCH = _ROWS_PER_W // _CB
_CIDX = _CB * _L
_GSUB = 1600
_NSUB = _CIDX // _GSUB


def _combine_rows(rows_v, w_v, b_v, out_v):
    def row_body(r, carry):
        base = r * _L
        acc0 = b_v[:]
        acc1 = b_v[:]
        for l in range(_L):
            wv = w_v[l, :]
            acc0 = acc0 + wv * rows_v[base + l, 0:_LANES]
            acc1 = acc1 + wv * rows_v[base + l, _LANES:_D]
        out_v[r, 0:_LANES] = jnp.maximum(acc0, 0.0)
        out_v[r, _LANES:_D] = jnp.maximum(acc1, 0.0)
        return carry

    lax.fori_loop(0, _CB, row_body, 0)


def _sc_body(enc_hbm, tab_hbm, w_hbm, b_hbm, out_hbm,
             idx0, idx1, rows0, rows1, out_v, w_v, b_v, sem0, sem1):
    wid = lax.axis_index("s") * _NC + lax.axis_index("c")
    row_base = wid * _ROWS_PER_W
    idx_base = row_base * _L

    pltpu.sync_copy(w_hbm, w_v)
    pltpu.sync_copy(b_hbm, b_v)

    bufs = ((idx0, rows0, sem0), (idx1, rows1, sem1))

    def start_chunk(c, idx_v, rows_v, sem):
        pltpu.sync_copy(enc_hbm.at[pl.ds(idx_base + c * _CIDX, _CIDX)], idx_v)
        for s in range(_NSUB):
            pltpu.async_copy(
                tab_hbm.at[idx_v.at[pl.ds(s * _GSUB, _GSUB)]],
                rows_v.at[pl.ds(s * _GSUB, _GSUB)],
                sem)

    def wait_chunk(rows_v, sem):
        pltpu.make_async_copy(tab_hbm.at[pl.ds(0, _CIDX)], rows_v, sem).wait()

    start_chunk(0, *bufs[0])
    start_chunk(1, *bufs[1])

    def outer(i, carry):
        for bsel in range(2):
            c = 2 * i + bsel
            idx_v, rows_v, sem = bufs[bsel]
            wait_chunk(rows_v, sem)
            _combine_rows(rows_v, w_v, b_v, out_v)
            pltpu.sync_copy(out_v, out_hbm.at[pl.ds(row_base + c * _CB, _CB)])

            @pl.when(c + 2 < _NCH)
            def _():
                start_chunk(c + 2, idx_v, rows_v, sem)

        return carry

    lax.fori_loop(0, _NCH // 2, outer, 0)


@jax.jit
def _run(enc_flat, table, w16, b16):
    mesh = plsc.VectorSubcoreMesh(core_axis_name="c", subcore_axis_name="s")
    sc = functools.partial(
        pl.kernel,
        out_type=jax.ShapeDtypeStruct((_B, _D), jnp.float32),
        mesh=mesh,
        scratch_types=[
            pltpu.VMEM((_CIDX,), jnp.int32),
            pltpu.VMEM((_CIDX,), jnp.int32),
            pltpu.VMEM((_CIDX, _D), jnp.float32),
            pltpu.VMEM((_CIDX, _D), jnp.float32),
            pltpu.VMEM((_CB, _D), jnp.float32),
            pltpu.VMEM((_L, _LANES), jnp.float32),
            pltpu.VMEM((_LANES,), jnp.float32),
            pltpu.SemaphoreType.DMA,
            pltpu.SemaphoreType.DMA,
        ],
        compiler_params=pltpu.CompilerParams(use_tc_tiling_on_sc=False),
    )(_sc_body)
    return sc(enc_flat, table, w16, b16)


def kernel(encoded, table, W, b):
    enc_flat = encoded.reshape(-1).astype(jnp.int32)
    w16 = jnp.broadcast_to(
        W.astype(jnp.float32).reshape(_L, 1), (_L, _LANES))
    b16 = jnp.broadcast_to(b.astype(jnp.float32).reshape(1), (_LANES,))
    tab_t = jax.lax.optimization_barrier(jnp.transpose(table))
    tab_lin = jnp.transpose(tab_t)
    return _run(enc_flat, tab_lin, w16, b16)
